# baseline (speedup 1.0000x reference)
import functools

import jax
import jax.numpy as jnp
from jax import lax
from jax.experimental import pallas as pl
from jax.experimental.pallas import tpu as pltpu
from jax.experimental.pallas import tpu_sc as plsc

E = 8
D = 768
F = 3072
S = 2048
T = 128
FC = 2
FCH = F // FC
NT = S // T + E
L = NT * T

_NC, _NS = 2, 16
NW = _NC * _NS
CHUNK = S // NW


def _gate_kernel(x_ref, gw_ref, gb_ref, pos_ref, meta_ref):
    x = x_ref[...]
    logits = jnp.dot(x, gw_ref[...], preferred_element_type=jnp.float32)
    logits = logits + gb_ref[...]
    idx8 = lax.broadcasted_iota(jnp.int32, (S, E), 1)
    m = jnp.max(logits, axis=1, keepdims=True)
    expert = jnp.min(jnp.where(logits == m, idx8, E), axis=1, keepdims=True)
    oh = (idx8 == expert).astype(jnp.int32)
    c = oh
    k = 1
    while k < S:
        c = c + jnp.concatenate(
            [jnp.zeros((k, E), jnp.int32), c[: S - k, :]], axis=0)
        k *= 2
    rank = jnp.sum(oh * (c - oh), axis=1, keepdims=True)
    counts = c[S - 1 : S, :]
    tiles = (counts + (T - 1)) // T
    tend = tiles
    k = 1
    while k < E:
        tend = tend + jnp.concatenate(
            [jnp.zeros((1, k), jnp.int32), tend[:, : E - k]], axis=1)
        k *= 2
    tstart_rows = (tend - tiles) * T
    pos_ref[...] = rank + jnp.sum(oh * tstart_rows, axis=1, keepdims=True)
    meta_ref[...] = jnp.concatenate(
        [tend, jnp.zeros((1, 64 - E), jnp.int32)], axis=1)


def _gate_route(x2, gate_W, gate_b):
    pos2, meta = pl.pallas_call(
        _gate_kernel,
        out_shape=[
            jax.ShapeDtypeStruct((S, 1), jnp.int32),
            jax.ShapeDtypeStruct((1, 64), jnp.int32),
        ],
    )(x2, gate_W, gate_b.reshape(1, E))
    return pos2.reshape(S), meta.reshape(64)


def _make_sc_permute(gather: bool, n_rows_out: int):
    mesh = plsc.VectorSubcoreMesh(
        core_axis_name="c", subcore_axis_name="s",
        num_cores=_NC, num_subcores=_NS)

    H = CHUNK // 2

    @functools.partial(
        pl.kernel,
        mesh=mesh,
        out_type=jax.ShapeDtypeStruct((n_rows_out, D), jnp.float32),
        scratch_types=[
            pltpu.VMEM((H,), jnp.int32),
            pltpu.VMEM((H,), jnp.int32),
            pltpu.VMEM((H, D), jnp.float32),
            pltpu.VMEM((H, D), jnp.float32),
            pltpu.SemaphoreType.DMA,
            pltpu.SemaphoreType.DMA,
            pltpu.SemaphoreType.DMA,
            pltpu.SemaphoreType.DMA,
        ],
    )
    def body(src_hbm, pos_hbm, out_hbm, idx0, idx1, buf0, buf1,
             s0, s1, s2, s3):
        wid = lax.axis_index("s") * _NC + lax.axis_index("c")
        base = wid * CHUNK
        pltpu.sync_copy(pos_hbm.at[pl.ds(base, H)], idx0)
        pltpu.sync_copy(pos_hbm.at[pl.ds(base + H, H)], idx1)
        if gather:
            c0 = pltpu.async_copy(src_hbm.at[idx0], buf0, s0)
            c1 = pltpu.async_copy(src_hbm.at[idx1], buf1, s1)
            c0.wait()
            o0 = pltpu.async_copy(buf0, out_hbm.at[pl.ds(base, H)], s2)
            c1.wait()
            o1 = pltpu.async_copy(buf1, out_hbm.at[pl.ds(base + H, H)], s3)
        else:
            c0 = pltpu.async_copy(src_hbm.at[pl.ds(base, H)], buf0, s0)
            c1 = pltpu.async_copy(src_hbm.at[pl.ds(base + H, H)], buf1, s1)
            c0.wait()
            o0 = pltpu.async_copy(buf0, out_hbm.at[idx0], s2)
            c1.wait()
            o1 = pltpu.async_copy(buf1, out_hbm.at[idx1], s3)
        o0.wait()
        o1.wait()

    return body


_sc_cache = {}


def _sc_scatter(x2, pos):
    if "scatter" not in _sc_cache:
        _sc_cache["scatter"] = _make_sc_permute(gather=False, n_rows_out=L)
    return _sc_cache["scatter"](x2, pos)


def _sc_gather(ys, pos):
    if "gather" not in _sc_cache:
        _sc_cache["gather"] = _make_sc_permute(gather=True, n_rows_out=S)
    return _sc_cache["gather"](ys, pos)


def _ffn_kernel(tend_ref, xs_ref, w1_ref, b1_ref, w2_ref, b2_ref, out_ref):
    e = pl.program_id(0)
    j = pl.program_id(1)
    t0 = jnp.where(e == 0, 0, tend_ref[jnp.maximum(e - 1, 0)])
    t1 = tend_ref[e]

    def tile(t, add):
        xt = xs_ref[pl.ds(t * T, T), :]
        h = jnp.dot(xt, w1_ref[0], preferred_element_type=jnp.float32)
        h = jnp.maximum(h + b1_ref[0], 0.0)
        p = jnp.dot(h, w2_ref[0], preferred_element_type=jnp.float32)
        if add:
            out_ref[pl.ds(t * T, T), :] += p
        else:
            out_ref[pl.ds(t * T, T), :] = p + b2_ref[0]
        return 0

    @pl.when(j == 0)
    def _():
        lax.fori_loop(t0, t1, lambda t, c: tile(t, False), 0)

    @pl.when(j != 0)
    def _():
        lax.fori_loop(t0, t1, lambda t, c: tile(t, True), 0)


def _ffn(tend, xs, W1, b1, W2, b2):
    grid_spec = pltpu.PrefetchScalarGridSpec(
        num_scalar_prefetch=1,
        grid=(E, FC),
        in_specs=[
            pl.BlockSpec((L, D), lambda e, j, td: (0, 0)),
            pl.BlockSpec((1, D, FCH), lambda e, j, td: (e, 0, j)),
            pl.BlockSpec((1, 1, FCH), lambda e, j, td: (e, 0, j)),
            pl.BlockSpec((1, FCH, D), lambda e, j, td: (e, j, 0)),
            pl.BlockSpec((1, 1, D), lambda e, j, td: (e, 0, 0)),
        ],
        out_specs=pl.BlockSpec((L, D), lambda e, j, td: (0, 0)),
    )
    return pl.pallas_call(
        _ffn_kernel,
        grid_spec=grid_spec,
        out_shape=jax.ShapeDtypeStruct((L, D), jnp.float32),
    )(tend, xs, W1, b1.reshape(E, 1, F), W2, b2.reshape(E, 1, D))


def kernel(x, gate_W, gate_b, W1, b1, W2, b2):
    x2 = x.reshape(S, D)
    pos, meta = _gate_route(x2, gate_W, gate_b)
    tend = meta[:E]
    xs = _sc_scatter(x2, pos)
    ys = _ffn(tend, xs, W1, b1, W2, b2)
    out = _sc_gather(ys, pos)
    return out.reshape(1, S, D)


# speedup vs baseline: 1.0055x; 1.0055x over previous
import functools

import jax
import jax.numpy as jnp
from jax import lax
from jax.experimental import pallas as pl
from jax.experimental.pallas import tpu as pltpu
from jax.experimental.pallas import tpu_sc as plsc

E = 8
D = 768
F = 3072
S = 2048
T = 128
FC = 2
FCH = F // FC
NT = S // T + E
L = NT * T

_NC, _NS = 2, 16
NW = _NC * _NS
CHUNK = S // NW


def _gate_kernel(x_ref, gw_ref, gb_ref, pos_ref, meta_ref):
    x = x_ref[...]
    logits = jnp.dot(x, gw_ref[...], preferred_element_type=jnp.float32)
    logits = logits + gb_ref[...]
    idx8 = lax.broadcasted_iota(jnp.int32, (S, E), 1)
    m = jnp.max(logits, axis=1, keepdims=True)
    expert = jnp.min(jnp.where(logits == m, idx8, E), axis=1, keepdims=True)
    oh = (idx8 == expert).astype(jnp.int32)
    c = oh
    k = 1
    while k < S:
        c = c + jnp.concatenate(
            [jnp.zeros((k, E), jnp.int32), c[: S - k, :]], axis=0)
        k *= 2
    rank = jnp.sum(oh * (c - oh), axis=1, keepdims=True)
    counts = c[S - 1 : S, :]
    tiles = (counts + (T - 1)) // T
    tend = tiles
    k = 1
    while k < E:
        tend = tend + jnp.concatenate(
            [jnp.zeros((1, k), jnp.int32), tend[:, : E - k]], axis=1)
        k *= 2
    tstart_rows = (tend - tiles) * T
    pos_ref[...] = rank + jnp.sum(oh * tstart_rows, axis=1, keepdims=True)
    meta_ref[...] = jnp.concatenate(
        [tend, jnp.zeros((1, 64 - E), jnp.int32)], axis=1)


def _gate_route(x2, gate_W, gate_b):
    pos2, meta = pl.pallas_call(
        _gate_kernel,
        out_shape=[
            jax.ShapeDtypeStruct((S, 1), jnp.int32),
            jax.ShapeDtypeStruct((1, 64), jnp.int32),
        ],
    )(x2, gate_W, gate_b.reshape(1, E))
    return pos2.reshape(S), meta.reshape(64)


def _make_sc_permute(gather: bool, n_rows_out: int):
    mesh = plsc.VectorSubcoreMesh(
        core_axis_name="c", subcore_axis_name="s",
        num_cores=_NC, num_subcores=_NS)

    @functools.partial(
        pl.kernel,
        mesh=mesh,
        out_type=jax.ShapeDtypeStruct((n_rows_out, D), jnp.float32),
        scratch_types=[
            pltpu.VMEM((CHUNK,), jnp.int32),
            pltpu.VMEM((CHUNK, D), jnp.float32),
            pltpu.SemaphoreType.DMA,
        ],
    )
    def body(src_hbm, pos_hbm, out_hbm, idx_v, rows_v, sem):
        wid = lax.axis_index("s") * _NC + lax.axis_index("c")
        base = wid * CHUNK
        pltpu.sync_copy(pos_hbm.at[pl.ds(base, CHUNK)], idx_v)
        if gather:
            pltpu.async_copy(src_hbm.at[idx_v], rows_v, sem).wait()
            pltpu.sync_copy(rows_v, out_hbm.at[pl.ds(base, CHUNK)])
        else:
            pltpu.sync_copy(src_hbm.at[pl.ds(base, CHUNK)], rows_v)
            pltpu.async_copy(rows_v, out_hbm.at[idx_v], sem).wait()

    return body


_sc_cache = {}


def _sc_scatter(x2, pos):
    if "scatter" not in _sc_cache:
        _sc_cache["scatter"] = _make_sc_permute(gather=False, n_rows_out=L)
    return _sc_cache["scatter"](x2, pos)


def _sc_gather(ys, pos):
    if "gather" not in _sc_cache:
        _sc_cache["gather"] = _make_sc_permute(gather=True, n_rows_out=S)
    return _sc_cache["gather"](ys, pos)


def _ffn_kernel(tend_ref, xs_ref, w1_ref, b1_ref, w2_ref, b2_ref, out_ref):
    e = pl.program_id(0)
    j = pl.program_id(1)
    t0 = jnp.where(e == 0, 0, tend_ref[jnp.maximum(e - 1, 0)])
    t1 = tend_ref[e]

    def tile(t, add):
        xt = xs_ref[pl.ds(t * T, T), :]
        h = jnp.dot(xt, w1_ref[0], preferred_element_type=jnp.float32)
        h = jnp.maximum(h + b1_ref[0], 0.0)
        p = jnp.dot(h, w2_ref[0], preferred_element_type=jnp.float32)
        if add:
            out_ref[pl.ds(t * T, T), :] += p
        else:
            out_ref[pl.ds(t * T, T), :] = p + b2_ref[0]
        return 0

    @pl.when(j == 0)
    def _():
        lax.fori_loop(t0, t1, lambda t, c: tile(t, False), 0)

    @pl.when(j != 0)
    def _():
        lax.fori_loop(t0, t1, lambda t, c: tile(t, True), 0)


def _ffn(tend, xs, W1, b1, W2, b2):
    grid_spec = pltpu.PrefetchScalarGridSpec(
        num_scalar_prefetch=1,
        grid=(E, FC),
        in_specs=[
            pl.BlockSpec((L, D), lambda e, j, td: (0, 0)),
            pl.BlockSpec((1, D, FCH), lambda e, j, td: (e, 0, j)),
            pl.BlockSpec((1, 1, FCH), lambda e, j, td: (e, 0, j)),
            pl.BlockSpec((1, FCH, D), lambda e, j, td: (e, j, 0)),
            pl.BlockSpec((1, 1, D), lambda e, j, td: (e, 0, 0)),
        ],
        out_specs=pl.BlockSpec((L, D), lambda e, j, td: (0, 0)),
    )
    return pl.pallas_call(
        _ffn_kernel,
        grid_spec=grid_spec,
        out_shape=jax.ShapeDtypeStruct((L, D), jnp.float32),
    )(tend, xs, W1, b1.reshape(E, 1, F), W2, b2.reshape(E, 1, D))


def kernel(x, gate_W, gate_b, W1, b1, W2, b2):
    x2 = x.reshape(S, D)
    pos, meta = _gate_route(x2, gate_W, gate_b)
    tend = meta[:E]
    xs = _sc_scatter(x2, pos)
    ys = _ffn(tend, xs, W1, b1, W2, b2)
    out = _sc_gather(ys, pos)
    return out.reshape(1, S, D)
